# baseline (speedup 1.0000x reference)
import jax
import jax.numpy as jnp
from jax.experimental import pallas as pl
from jax.experimental.pallas import tpu as pltpu

EPS = 1e-05
INV_D = 30.0
LOG2E = 1.4426950408889634
H = W = 16
SKIP = 2
N_KEEP = (H - 2 * SKIP) * (W - 2 * SKIP)
CK = 2048


def _moca_kernel(x_ref, emb_ref, codes_ref, bow_ref, e_ref, s_ref):
    b = pl.program_id(0)
    cur = jax.lax.rem(b, 2)
    prv = 1 - cur
    L = x_ref.shape[1]
    K = codes_ref.shape[2]

    xv = x_ref[0]
    n = jnp.sqrt(jnp.sum(xv * xv, axis=1, keepdims=True))
    xb = (xv * ((INV_D * LOG2E) / jnp.maximum(n, EPS))).astype(jnp.bfloat16)
    s = jnp.zeros((L, 1), jnp.float32)
    for k in range(K // CK):
        acc = jax.lax.dot_general(
            xb, emb_ref[pl.ds(k * CK, CK), :],
            dimension_numbers=(((1,), (1,)), ((), ())),
            preferred_element_type=jnp.float32,
        )
        e = jnp.exp2(acc - (INV_D * LOG2E))
        e_ref[pl.ds(cur, 1), :, pl.ds(k * CK, CK)] = e[None]
        s = s + jnp.sum(e, axis=1, keepdims=True)
    s_ref[pl.ds(cur, 1)] = s[None]

    t = jax.lax.broadcasted_iota(jnp.int32, (1, L), 1)
    tr = t // W
    tc = t % W
    keep = (tr >= SKIP) & (tr < H - SKIP) & (tc >= SKIP) & (tc < W - SKIP)
    mrow = jnp.where(keep, 1.0 / N_KEEP, 0.0)

    sp = s_ref[pl.ds(prv, 1)][0]
    r = 1.0 / sp
    w = mrow * r.reshape(1, L)
    bow_parts = []
    for k in range(K // CK):
        ek = e_ref[pl.ds(prv, 1), :, pl.ds(k * CK, CK)][0]
        codes_ref[0, :, pl.ds(k * CK, CK)] = ek * r
        bow_parts.append(jax.lax.dot_general(
            w, ek,
            dimension_numbers=(((1,), (0,)), ((), ())),
            preferred_element_type=jnp.float32,
        ))
    bow = jnp.concatenate(bow_parts, axis=1)
    l1 = jnp.sum(jnp.abs(bow))
    bow_ref[0] = bow * (1.0 / jnp.maximum(l1, EPS))


@jax.jit
def kernel(x, embedding):
    B = x.shape[0]
    xs = x[:, 1:, :]
    L = xs.shape[1]
    D = xs.shape[2]
    K = embedding.shape[0]
    embedding = embedding.astype(jnp.bfloat16)
    codes, bow = pl.pallas_call(
        _moca_kernel,
        grid=(B + 1,),
        in_specs=[
            pl.BlockSpec((1, L, D), lambda b: (jnp.minimum(b, B - 1), 0, 0)),
            pl.BlockSpec((K, D), lambda b: (0, 0)),
        ],
        out_specs=[
            pl.BlockSpec((1, L, K), lambda b: (jnp.maximum(b - 1, 0), 0, 0)),
            pl.BlockSpec((1, 1, K), lambda b: (jnp.maximum(b - 1, 0), 0, 0)),
        ],
        out_shape=[
            jax.ShapeDtypeStruct((B, L, K), jnp.float32),
            jax.ShapeDtypeStruct((B, 1, K), jnp.float32),
        ],
        scratch_shapes=[
            pltpu.VMEM((2, L, K), jnp.float32),
            pltpu.VMEM((2, L, 1), jnp.float32),
        ],
    )(xs, embedding)
    return (bow.reshape(B, K), codes)


# speedup vs baseline: 1.0623x; 1.0623x over previous
import jax
import jax.numpy as jnp
from jax.experimental import pallas as pl
from jax.experimental.pallas import tpu as pltpu

EPS = 1e-05
INV_D = 30.0
LOG2E = 1.4426950408889634
H = W = 16
SKIP = 2
N_KEEP = (H - 2 * SKIP) * (W - 2 * SKIP)
CK = 1024


def _moca_kernel(x_ref, emb_ref, codes_ref, bow_ref, e0_ref, e1_ref):
    L = x_ref.shape[1]
    K = codes_ref.shape[2]
    t = jax.lax.broadcasted_iota(jnp.int32, (1, L), 1)
    tr = t // W
    tc = t % W
    keep = (tr >= SKIP) & (tr < H - SKIP) & (tc >= SKIP) & (tc < W - SKIP)
    mrow = jnp.where(keep, 1.0 / N_KEEP, 0.0)

    def pass_a(half, e_refh):
        xv = x_ref[half].astype(jnp.float32)
        n = jnp.sqrt(jnp.sum(xv * xv, axis=1, keepdims=True))
        xb = (xv * ((INV_D * LOG2E) / jnp.maximum(n, EPS))).astype(jnp.bfloat16)
        s = jnp.zeros((L, 1), jnp.float32)
        for k in range(K // CK):
            acc = jax.lax.dot_general(
                xb, emb_ref[pl.ds(k * CK, CK), :],
                dimension_numbers=(((1,), (1,)), ((), ())),
                preferred_element_type=jnp.float32,
            )
            e = jnp.exp2(acc - (INV_D * LOG2E))
            e_refh[:, pl.ds(k * CK, CK)] = e.astype(jnp.bfloat16)
            s = s + jnp.sum(e, axis=1, keepdims=True)
        return s

    def pass_b(half, e_refh, s):
        r = 1.0 / s
        w = (mrow * r.reshape(1, L)).astype(jnp.bfloat16)
        bow_parts = []
        for k in range(K // CK):
            ek = e_refh[:, pl.ds(k * CK, CK)]
            codes_ref[half, :, pl.ds(k * CK, CK)] = ek.astype(jnp.float32) * r
            bow_parts.append(jax.lax.dot_general(
                w, ek,
                dimension_numbers=(((1,), (0,)), ((), ())),
                preferred_element_type=jnp.float32,
            ))
        bow = jnp.concatenate(bow_parts, axis=1)
        l1 = jnp.sum(jnp.abs(bow))
        bow_ref[half] = bow * (1.0 / jnp.maximum(l1, EPS))

    s0 = pass_a(0, e0_ref)
    pass_b(0, e0_ref, s0)
    s1 = pass_a(1, e1_ref)
    pass_b(1, e1_ref, s1)


@jax.jit
def kernel(x, embedding):
    B = x.shape[0]
    xs = x[:, 1:, :].astype(jnp.bfloat16)
    L = xs.shape[1]
    D = xs.shape[2]
    K = embedding.shape[0]
    embedding = embedding.astype(jnp.bfloat16)
    codes, bow = pl.pallas_call(
        _moca_kernel,
        grid=(B // 2,),
        in_specs=[
            pl.BlockSpec((2, L, D), lambda b: (b, 0, 0)),
            pl.BlockSpec((K, D), lambda b: (0, 0)),
        ],
        out_specs=[
            pl.BlockSpec((2, L, K), lambda b: (b, 0, 0)),
            pl.BlockSpec((2, 1, K), lambda b: (b, 0, 0)),
        ],
        out_shape=[
            jax.ShapeDtypeStruct((B, L, K), jnp.float32),
            jax.ShapeDtypeStruct((B, 1, K), jnp.float32),
        ],
        scratch_shapes=[
            pltpu.VMEM((L, K), jnp.bfloat16),
            pltpu.VMEM((L, K), jnp.bfloat16),
        ],
    )(xs, embedding)
    return (bow.reshape(B, K), codes)
